# Initial kernel scaffold; baseline (speedup 1.0000x reference)
import functools

import jax
import jax.numpy as jnp
from jax.experimental import pallas as pl
from jax.experimental.pallas import tpu as pltpu

_EPS = 1e-5
_ROWS = 8


def _conv_kernel(xa_ref, xb_ref, w_ref, b_ref, psum_ref, psq_ref, g_ref,
                 bb_ref, out_ref, osum_ref, osq_ref, *, apply_bn, relu,
                 count, W):
    i = pl.program_id(1)
    rows = jnp.concatenate([xa_ref[0], xb_ref[0][:2]], axis=0)
    if apply_bn:
        n = jnp.float32(count)
        s = psum_ref[0, :]
        q = psq_ref[0, :]
        mean = s / n
        var = q / n - mean * mean
        inv = jax.lax.rsqrt(var + _EPS)
        rows = (rows - mean[None, None, :]) * (inv * g_ref[0, :])[None, None, :] \
            + bb_ref[0, :][None, None, :]
        if relu:
            rows = jnp.maximum(rows, 0.0)
        r_ids = i * _ROWS + jax.lax.broadcasted_iota(jnp.int32, rows.shape, 0)
        c_ids = jax.lax.broadcasted_iota(jnp.int32, rows.shape, 1)
        H = pl.num_programs(1) * _ROWS
        valid = (r_ids >= 1) & (r_ids <= H) & (c_ids >= 1) & (c_ids <= W)
        rows = jnp.where(valid, rows, 0.0)
    cout = out_ref.shape[-1]
    acc = jnp.zeros((_ROWS, W, cout), jnp.float32)
    for dh in range(3):
        for dw in range(3):
            patch = rows[dh:dh + _ROWS, dw:dw + W, :]
            acc = acc + jax.lax.dot_general(
                patch, w_ref[dh, dw],
                dimension_numbers=(((2,), (0,)), ((), ())),
                preferred_element_type=jnp.float32)
    y = acc + b_ref[0, :][None, None, :]
    out_ref[0] = y

    @pl.when((pl.program_id(0) == 0) & (i == 0))
    def _():
        osum_ref[...] = jnp.zeros_like(osum_ref)
        osq_ref[...] = jnp.zeros_like(osq_ref)

    osum_ref[0:1, :] += jnp.sum(y, axis=(0, 1))[None, :]
    osq_ref[0:1, :] += jnp.sum(y * y, axis=(0, 1))[None, :]


def _conv_layer(xp, w, b, psum, psq, g, bb, *, apply_bn, relu):
    B, Hp, Wp, Cin = xp.shape
    H, W = Hp - 8, Wp - 2
    nb = H // _ROWS
    Cout = w.shape[-1]
    count = B * H * W
    kfn = functools.partial(_conv_kernel, apply_bn=apply_bn, relu=relu,
                            count=count, W=W)
    blk_in = (1, _ROWS, Wp, Cin)
    return pl.pallas_call(
        kfn,
        grid=(B, nb),
        in_specs=[
            pl.BlockSpec(blk_in, lambda bidx, i: (bidx, i, 0, 0)),
            pl.BlockSpec(blk_in, lambda bidx, i: (bidx, i + 1, 0, 0)),
            pl.BlockSpec((3, 3, Cin, Cout), lambda bidx, i: (0, 0, 0, 0)),
            pl.BlockSpec((1, Cout), lambda bidx, i: (0, 0)),
            pl.BlockSpec((8, Cin), lambda bidx, i: (0, 0)),
            pl.BlockSpec((8, Cin), lambda bidx, i: (0, 0)),
            pl.BlockSpec((1, Cin), lambda bidx, i: (0, 0)),
            pl.BlockSpec((1, Cin), lambda bidx, i: (0, 0)),
        ],
        out_specs=[
            pl.BlockSpec((1, _ROWS, W, Cout), lambda bidx, i: (bidx, i, 0, 0)),
            pl.BlockSpec((8, Cout), lambda bidx, i: (0, 0)),
            pl.BlockSpec((8, Cout), lambda bidx, i: (0, 0)),
        ],
        out_shape=[
            jax.ShapeDtypeStruct((B, H, W, Cout), jnp.float32),
            jax.ShapeDtypeStruct((8, Cout), jnp.float32),
            jax.ShapeDtypeStruct((8, Cout), jnp.float32),
        ],
    )(xp, xp, w, b, psum, psq, g, bb)


def _kmeans_kernel(y_ref, psum_ref, psq_ref, g_ref, bb_ref, init_ref, fb_ref,
                   out_ref, x_s, xsq_s, oh_s, *, count, n_iters, tol, K):
    N, C = x_s.shape
    n = jnp.float32(count)
    mean = psum_ref[0, :] / n
    var = psq_ref[0, :] / n - mean * mean
    inv = jax.lax.rsqrt(var + _EPS)
    X = (y_ref[0] - mean[None, :]) * (inv * g_ref[0, :])[None, :] \
        + bb_ref[0, :][None, :]
    x_s[...] = X
    xsq = jnp.sum(X * X, axis=1, keepdims=True)
    xsq_s[...] = jnp.broadcast_to(xsq, (N, 8))

    def gather5(idx_fn):
        parts = [x_s[pl.ds(idx_fn(k), 1), :] for k in range(K)]
        parts.append(jnp.zeros((8 - K, C), jnp.float32))
        return jnp.concatenate(parts, axis=0)

    c0 = gather5(lambda k: init_ref[0, k])

    lane = jax.lax.broadcasted_iota(jnp.int32, (N, 8), 1)
    dummy = lane >= K

    def body(carry):
        it, _, Cc = carry
        c_sq = jnp.sum(Cc * Cc, axis=1)
        prod = jax.lax.dot_general(
            x_s[...], Cc, dimension_numbers=(((1,), (1,)), ((), ())),
            preferred_element_type=jnp.float32)
        d2 = jnp.maximum(xsq_s[...] + c_sq[None, :] - 2.0 * prod, 0.0)
        d2 = jnp.where(dummy, jnp.float32(3e38), d2)
        mind = jnp.min(d2, axis=1, keepdims=True)
        first = jnp.min(jnp.where(d2 <= mind, lane, 8), axis=1, keepdims=True)
        onehot = (lane == first).astype(jnp.float32)
        oh_s[...] = onehot
        sums = jax.lax.dot_general(
            onehot, x_s[...], dimension_numbers=(((0,), (0,)), ((), ())),
            preferred_element_type=jnp.float32)
        counts = jnp.sum(onehot, axis=0)
        means = sums / jnp.maximum(counts, 1.0)[:, None]
        fbX = gather5(lambda k: fb_ref[0, it, k])
        newC = jnp.where((counts > 0.0)[:, None], means, fbX)
        shift = jnp.sum(jnp.sqrt(jnp.sum((newC - Cc) ** 2, axis=1)))
        new_done = shift <= tol
        Cn = jnp.where(new_done, Cc, newC)
        return it + 1, new_done, Cn

    def cond(carry):
        it, done, _ = carry
        return (it < n_iters) & jnp.logical_not(done)

    jax.lax.while_loop(cond, body, (jnp.int32(0), jnp.bool_(False), c0))

    onehot = oh_s[...]
    sums = jax.lax.dot_general(
        onehot, x_s[...], dimension_numbers=(((0,), (0,)), ((), ())),
        preferred_element_type=jnp.float32)
    counts = jnp.sum(onehot, axis=0)
    means = sums / jnp.maximum(counts, 1.0)[:, None]
    out_ref[0] = jax.lax.dot_general(
        onehot, means, dimension_numbers=(((1,), (0,)), ((), ())),
        preferred_element_type=jnp.float32)


def _kmeans(y3, psum, psq, g, bb, init_idx, fb_idx, *, n_iters=100, tol=1e-4,
            K=5):
    B, N, C = y3.shape
    count = B * N
    kfn = functools.partial(_kmeans_kernel, count=count, n_iters=n_iters,
                            tol=tol, K=K)
    return pl.pallas_call(
        kfn,
        grid=(B,),
        in_specs=[
            pl.BlockSpec((1, N, C), lambda b: (b, 0, 0)),
            pl.BlockSpec((8, C), lambda b: (0, 0)),
            pl.BlockSpec((8, C), lambda b: (0, 0)),
            pl.BlockSpec((1, C), lambda b: (0, 0)),
            pl.BlockSpec((1, C), lambda b: (0, 0)),
            pl.BlockSpec((1, K), lambda b: (b, 0), memory_space=pltpu.SMEM),
            pl.BlockSpec((1, n_iters, K), lambda b: (b, 0, 0),
                         memory_space=pltpu.SMEM),
        ],
        out_specs=pl.BlockSpec((1, N, C), lambda b: (b, 0, 0)),
        out_shape=jax.ShapeDtypeStruct((B, N, C), jnp.float32),
        scratch_shapes=[
            pltpu.VMEM((N, C), jnp.float32),
            pltpu.VMEM((N, 8), jnp.float32),
            pltpu.VMEM((N, 8), jnp.float32),
        ],
    )(y3, psum, psq, g, bb, init_idx, fb_idx)


def _pad(y):
    return jnp.pad(y, ((0, 0), (1, 7), (1, 1), (0, 0)))


def kernel(x, w1, b1, g1, bb1, w2, b2, g2, bb2, w3, b3, g3, bb3):
    B, Cin, H, W = x.shape
    K, n_iters = 5, 100

    def wt(w):
        return jnp.transpose(w, (2, 3, 1, 0))

    def row(v):
        return v.reshape(1, -1).astype(jnp.float32)

    zeros_in = jnp.zeros((8, Cin), jnp.float32)
    xp = _pad(jnp.transpose(x, (0, 2, 3, 1)))
    y1, s1, q1 = _conv_layer(xp, wt(w1), row(b1), zeros_in, zeros_in,
                             row(jnp.ones((Cin,))), row(jnp.zeros((Cin,))),
                             apply_bn=False, relu=False)
    y2, s2, q2 = _conv_layer(_pad(y1), wt(w2), row(b2), s1, q1, row(g1),
                             row(bb1), apply_bn=True, relu=True)
    y3, s3, q3 = _conv_layer(_pad(y2), wt(w3), row(b3), s2, q2, row(g2),
                             row(bb2), apply_bn=True, relu=True)

    C3 = y3.shape[-1]
    N = H * W
    init_list, fb_list = [], []
    for i in range(B):
        ki = jax.random.fold_in(jax.random.key(42), i)
        init_list.append(jax.random.permutation(ki, N)[:K].astype(jnp.int32))
        fb_list.append(jax.random.randint(
            jax.random.fold_in(ki, 1), (n_iters, K), 0, N).astype(jnp.int32))
    init_idx = jnp.stack(init_list)
    fb_idx = jnp.stack(fb_list)

    out = _kmeans(y3.reshape(B, N, C3), s3, q3, row(g3), row(bb3),
                  init_idx, fb_idx, n_iters=n_iters, K=K)
    return jnp.transpose(out, (0, 2, 1)).reshape(B, C3, H, W)


# speedup vs baseline: 16.0380x; 16.0380x over previous
import functools

import jax
import jax.numpy as jnp
from jax.experimental import pallas as pl
from jax.experimental.pallas import tpu as pltpu

_EPS = 1e-5
_ROWS = 8


def _conv_kernel(xa_ref, xb_ref, w_ref, b_ref, psum_ref, psq_ref, g_ref,
                 bb_ref, out_ref, osum_ref, osq_ref, *, apply_bn, relu,
                 count, W):
    i = pl.program_id(1)
    rows = jnp.concatenate([xa_ref[0], xb_ref[0][:2]], axis=0)
    if apply_bn:
        n = jnp.float32(count)
        s = psum_ref[0, :]
        q = psq_ref[0, :]
        mean = s / n
        var = q / n - mean * mean
        inv = jax.lax.rsqrt(var + _EPS)
        rows = (rows - mean[None, None, :]) * (inv * g_ref[0, :])[None, None, :] \
            + bb_ref[0, :][None, None, :]
        if relu:
            rows = jnp.maximum(rows, 0.0)
        r_ids = i * _ROWS + jax.lax.broadcasted_iota(jnp.int32, rows.shape, 0)
        c_ids = jax.lax.broadcasted_iota(jnp.int32, rows.shape, 1)
        H = pl.num_programs(1) * _ROWS
        valid = (r_ids >= 1) & (r_ids <= H) & (c_ids >= 1) & (c_ids <= W)
        rows = jnp.where(valid, rows, 0.0)
    cout = out_ref.shape[-1]
    acc = jnp.zeros((_ROWS, W, cout), jnp.float32)
    for dh in range(3):
        for dw in range(3):
            patch = rows[dh:dh + _ROWS, dw:dw + W, :]
            acc = acc + jax.lax.dot_general(
                patch, w_ref[dh, dw],
                dimension_numbers=(((2,), (0,)), ((), ())),
                preferred_element_type=jnp.float32)
    y = acc + b_ref[0, :][None, None, :]
    out_ref[0] = y

    @pl.when((pl.program_id(0) == 0) & (i == 0))
    def _():
        osum_ref[...] = jnp.zeros_like(osum_ref)
        osq_ref[...] = jnp.zeros_like(osq_ref)

    osum_ref[0:1, :] += jnp.sum(y, axis=(0, 1))[None, :]
    osq_ref[0:1, :] += jnp.sum(y * y, axis=(0, 1))[None, :]


def _conv_layer(xp, w, b, psum, psq, g, bb, *, apply_bn, relu):
    B, Hp, Wp, Cin = xp.shape
    H, W = Hp - 8, Wp - 2
    nb = H // _ROWS
    Cout = w.shape[-1]
    count = B * H * W
    kfn = functools.partial(_conv_kernel, apply_bn=apply_bn, relu=relu,
                            count=count, W=W)
    blk_in = (1, _ROWS, Wp, Cin)
    return pl.pallas_call(
        kfn,
        grid=(B, nb),
        in_specs=[
            pl.BlockSpec(blk_in, lambda bidx, i: (bidx, i, 0, 0)),
            pl.BlockSpec(blk_in, lambda bidx, i: (bidx, i + 1, 0, 0)),
            pl.BlockSpec((3, 3, Cin, Cout), lambda bidx, i: (0, 0, 0, 0)),
            pl.BlockSpec((1, Cout), lambda bidx, i: (0, 0)),
            pl.BlockSpec((8, Cin), lambda bidx, i: (0, 0)),
            pl.BlockSpec((8, Cin), lambda bidx, i: (0, 0)),
            pl.BlockSpec((1, Cin), lambda bidx, i: (0, 0)),
            pl.BlockSpec((1, Cin), lambda bidx, i: (0, 0)),
        ],
        out_specs=[
            pl.BlockSpec((1, _ROWS, W, Cout), lambda bidx, i: (bidx, i, 0, 0)),
            pl.BlockSpec((8, Cout), lambda bidx, i: (0, 0)),
            pl.BlockSpec((8, Cout), lambda bidx, i: (0, 0)),
        ],
        out_shape=[
            jax.ShapeDtypeStruct((B, H, W, Cout), jnp.float32),
            jax.ShapeDtypeStruct((8, Cout), jnp.float32),
            jax.ShapeDtypeStruct((8, Cout), jnp.float32),
        ],
    )(xp, xp, w, b, psum, psq, g, bb)


def _kmeans_kernel(yt_ref, bnp_ref, init_ref, fb_ref, out_ref, xt_s, xtb_s,
                   oh_s, *, count, n_iters, tol, K):
    C, N = xt_s.shape
    n = jnp.float32(count)
    mean = bnp_ref[:, 0:1] / n
    var = bnp_ref[:, 1:2] / n - mean * mean
    scale = jax.lax.rsqrt(var + _EPS) * bnp_ref[:, 2:3]
    xt_s[...] = (yt_ref[0] - mean) * scale + bnp_ref[:, 3:4]
    xtb_s[...] = xt_s[...].astype(jnp.bfloat16)

    lane = jax.lax.broadcasted_iota(jnp.int32, (1, N), 1)
    subl = jax.lax.broadcasted_iota(jnp.int32, (8, N), 0)
    subl8 = jax.lax.broadcasted_iota(jnp.int32, (8, 1), 0)

    def gather_cols(idx_fn):
        cols = []
        for k in range(K):
            m = (lane == idx_fn(k)).astype(jnp.float32)
            cols.append(jnp.sum(xt_s[...] * m, axis=1, keepdims=True))
        cols.append(jnp.zeros((C, 8 - K), jnp.float32))
        return jnp.concatenate(cols, axis=1)

    c0 = gather_cols(lambda k: init_ref[0, 0, k])

    def body(carry):
        it, _, Ct = carry
        csq = jnp.zeros((8, 1), jnp.float32)
        for k in range(K):
            csq = jnp.where(subl8 == k, jnp.sum(Ct[:, k:k + 1] ** 2), csq)
        prod = jax.lax.dot_general(
            Ct.astype(jnp.bfloat16), xtb_s[...],
            dimension_numbers=(((0,), (0,)), ((), ())),
            preferred_element_type=jnp.float32)
        d2 = csq - 2.0 * prod
        d2 = jnp.where(subl >= K, jnp.float32(3e38), d2)
        mind = jnp.min(d2, axis=0, keepdims=True)
        first = jnp.min(jnp.where(d2 <= mind, subl, 8), axis=0, keepdims=True)
        oh = (subl == first).astype(jnp.float32)
        oh_s[...] = oh
        counts = jnp.sum(oh, axis=1, keepdims=True)
        need_fb = jnp.any(counts[:K, 0] < 0.5)
        fb = jax.lax.cond(
            need_fb, lambda: gather_cols(lambda k: fb_ref[0, it, k]),
            lambda: jnp.zeros((C, 8), jnp.float32))
        cols = []
        for k in range(K):
            s_k = jnp.sum(xt_s[...] * oh[k:k + 1, :], axis=1, keepdims=True)
            cnt = counts[k, 0]
            m_k = s_k / jnp.maximum(cnt, 1.0)
            cols.append(jnp.where(cnt > 0.0, m_k, fb[:, k:k + 1]))
        cols.append(jnp.zeros((C, 8 - K), jnp.float32))
        newCt = jnp.concatenate(cols, axis=1)
        shift = jnp.sum(jnp.sqrt(jnp.sum((newCt - Ct) ** 2, axis=0)))
        new_done = shift <= tol
        Cn = jnp.where(new_done, Ct, newCt)
        return it + 1, new_done, Cn

    def cond(carry):
        it, done, _ = carry
        return (it < n_iters) & jnp.logical_not(done)

    jax.lax.while_loop(cond, body, (jnp.int32(0), jnp.bool_(False), c0))

    oh = oh_s[...]
    counts = jnp.sum(oh, axis=1, keepdims=True)
    cols = []
    for k in range(K):
        s_k = jnp.sum(xt_s[...] * oh[k:k + 1, :], axis=1, keepdims=True)
        cols.append(s_k / jnp.maximum(counts[k, 0], 1.0))
    out = jnp.zeros((C, N), jnp.float32)
    for k in range(K):
        out = out + cols[k] * oh[k:k + 1, :]
    out_ref[0] = out


def _kmeans(y3t, bnp, init_idx, fb_idx, *, n_iters=100, tol=1e-4, K=5):
    B, C, N = y3t.shape
    count = B * N
    kfn = functools.partial(_kmeans_kernel, count=count, n_iters=n_iters,
                            tol=tol, K=K)
    return pl.pallas_call(
        kfn,
        grid=(B,),
        in_specs=[
            pl.BlockSpec((1, C, N), lambda b: (b, 0, 0)),
            pl.BlockSpec((C, 8), lambda b: (0, 0)),
            pl.BlockSpec((1, 1, K), lambda b: (b, 0, 0),
                         memory_space=pltpu.SMEM),
            pl.BlockSpec((1, n_iters, K), lambda b: (b, 0, 0),
                         memory_space=pltpu.SMEM),
        ],
        out_specs=pl.BlockSpec((1, C, N), lambda b: (b, 0, 0)),
        out_shape=jax.ShapeDtypeStruct((B, C, N), jnp.float32),
        scratch_shapes=[
            pltpu.VMEM((C, N), jnp.float32),
            pltpu.VMEM((C, N), jnp.bfloat16),
            pltpu.VMEM((8, N), jnp.float32),
        ],
    )(y3t, bnp, init_idx, fb_idx)


def _pad(y):
    return jnp.pad(y, ((0, 0), (1, 7), (1, 1), (0, 0)))


def kernel(x, w1, b1, g1, bb1, w2, b2, g2, bb2, w3, b3, g3, bb3):
    B, Cin, H, W = x.shape
    K, n_iters = 5, 100

    def wt(w):
        return jnp.transpose(w, (2, 3, 1, 0))

    def row(v):
        return v.reshape(1, -1).astype(jnp.float32)

    zeros_in = jnp.zeros((8, Cin), jnp.float32)
    xp = _pad(jnp.transpose(x, (0, 2, 3, 1)))
    y1, s1, q1 = _conv_layer(xp, wt(w1), row(b1), zeros_in, zeros_in,
                             row(jnp.ones((Cin,))), row(jnp.zeros((Cin,))),
                             apply_bn=False, relu=False)
    y2, s2, q2 = _conv_layer(_pad(y1), wt(w2), row(b2), s1, q1, row(g1),
                             row(bb1), apply_bn=True, relu=True)
    y3, s3, q3 = _conv_layer(_pad(y2), wt(w3), row(b3), s2, q2, row(g2),
                             row(bb2), apply_bn=True, relu=True)

    C3 = y3.shape[-1]
    N = H * W
    init_list, fb_list = [], []
    for i in range(B):
        ki = jax.random.fold_in(jax.random.key(42), i)
        init_list.append(jax.random.permutation(ki, N)[:K].astype(jnp.int32))
        fb_list.append(jax.random.randint(
            jax.random.fold_in(ki, 1), (n_iters, K), 0, N).astype(jnp.int32))
    init_idx = jnp.stack(init_list).reshape(B, 1, K)
    fb_idx = jnp.stack(fb_list)

    bnp = jnp.concatenate(
        [s3[0:1], q3[0:1], row(g3), row(bb3), jnp.zeros((4, C3))], axis=0).T
    y3t = jnp.transpose(y3.reshape(B, N, C3), (0, 2, 1))
    out = _kmeans(y3t, bnp, init_idx, fb_idx, n_iters=n_iters, K=K)
    return out.reshape(B, C3, H, W)
